# baseline (speedup 1.0000x reference)
import functools

import jax
import jax.numpy as jnp
from jax import lax
from jax.experimental import pallas as pl
from jax.experimental.pallas import tpu as pltpu
from jax.experimental.pallas import tpu_sc as plsc

D_MODEL = 1024
L_SEQ = 4096
NUM_EMB = 5
TEXT_ID = 0
VISUAL_ID = 3

NUM_CORES = 2
NUM_SUBCORES = 16
LANES = 16
NUM_WORKERS = NUM_CORES * NUM_SUBCORES
ROWS_PER_WORKER = L_SEQ // NUM_WORKERS
CHUNK = 16
NCHUNK = ROWS_PER_WORKER // CHUNK
SLICES = D_MODEL // LANES

_MESH = plsc.VectorSubcoreMesh(core_axis_name="c", subcore_axis_name="s")


@functools.partial(
    pl.kernel,
    out_type=jax.ShapeDtypeStruct((L_SEQ, 1, D_MODEL), jnp.float32),
    mesh=_MESH,
    scratch_types=[
        pltpu.VMEM((NUM_EMB, D_MODEL), jnp.float32),
        pltpu.VMEM((LANES,), jnp.int32),
        pltpu.VMEM((CHUNK, 1, D_MODEL), jnp.float32),
        pltpu.VMEM((CHUNK, 1, D_MODEL), jnp.float32),
        pltpu.SemaphoreType.DMA,
        pltpu.SemaphoreType.DMA,
        pltpu.SemaphoreType.DMA,
    ],
)
def _emb_lookup(table_hbm, ntxt_hbm, out_hbm, tab_v, ntxt_v, buf0, buf1,
                sem0, sem1, sem_in):
    wid = lax.axis_index("s") * NUM_CORES + lax.axis_index("c")
    base = wid * ROWS_PER_WORKER
    tab_dma = pltpu.async_copy(table_hbm, tab_v, sem_in)
    pltpu.sync_copy(ntxt_hbm, ntxt_v)
    ntxt = ntxt_v[...]
    tab_dma.wait()

    def fill(buf, cbase, nrows):
        conds = [jnp.full((LANES,), cbase + r, jnp.int32) < ntxt
                 for r in range(nrows)]

        def body(s, carry):
            off = s * LANES
            e0 = tab_v[TEXT_ID, pl.ds(off, LANES)]
            e3 = tab_v[VISUAL_ID, pl.ds(off, LANES)]
            for r in range(nrows):
                buf[r, 0, pl.ds(off, LANES)] = jnp.where(conds[r], e0, e3)
            return carry
        lax.fori_loop(0, SLICES, body, 0)

    sizes = (4, 4, 8) + (CHUNK,) * ((ROWS_PER_WORKER - 16) // CHUNK)
    bufs = (buf0, buf1)
    sems = (sem0, sem1)
    handles = [None, None]
    off_rows = 0
    for c, n in enumerate(sizes):
        b = c % 2
        if handles[b] is not None:
            handles[b].wait()
        fill(bufs[b], base + off_rows, n)
        handles[b] = pltpu.async_copy(
            bufs[b].at[pl.ds(0, n)],
            out_hbm.at[pl.ds(base + off_rows, n)], sems[b])
        off_rows += n
    for h in handles:
        if h is not None:
            h.wait()


def kernel(x, num_frame, embedding):
    L, N, D = x.shape
    num_txt = jnp.full((LANES,), L - num_frame, dtype=jnp.int32)
    return _emb_lookup(embedding, num_txt)


# speedup vs baseline: 1.0404x; 1.0404x over previous
import functools

import jax
import jax.numpy as jnp
from jax import lax
from jax.experimental import pallas as pl
from jax.experimental.pallas import tpu as pltpu
from jax.experimental.pallas import tpu_sc as plsc

D_MODEL = 1024
L_SEQ = 4096
NUM_EMB = 5
TEXT_ID = 0
VISUAL_ID = 3

NUM_CORES = 2
NUM_SUBCORES = 16
LANES = 16
NUM_WORKERS = NUM_CORES * NUM_SUBCORES
ROWS_PER_WORKER = L_SEQ // NUM_WORKERS
CHUNK = 16
NCHUNK = ROWS_PER_WORKER // CHUNK
SLICES = D_MODEL // LANES

_MESH = plsc.VectorSubcoreMesh(core_axis_name="c", subcore_axis_name="s")


@functools.partial(
    pl.kernel,
    out_type=jax.ShapeDtypeStruct((L_SEQ, 1, D_MODEL), jnp.float32),
    mesh=_MESH,
    scratch_types=[
        pltpu.VMEM((NUM_EMB, D_MODEL), jnp.float32),
        pltpu.VMEM((LANES,), jnp.int32),
        pltpu.VMEM((CHUNK, 1, D_MODEL), jnp.float32),
        pltpu.VMEM((CHUNK, 1, D_MODEL), jnp.float32),
        pltpu.SemaphoreType.DMA,
        pltpu.SemaphoreType.DMA,
        pltpu.SemaphoreType.DMA,
    ],
)
def _emb_lookup(table_hbm, ntxt_hbm, out_hbm, tab_v, ntxt_v, buf0, buf1,
                sem0, sem1, sem_in):
    wid = lax.axis_index("s") * NUM_CORES + lax.axis_index("c")
    base = wid * ROWS_PER_WORKER
    tab_dma = pltpu.async_copy(table_hbm, tab_v, sem_in)
    pltpu.sync_copy(ntxt_hbm, ntxt_v)
    ntxt = ntxt_v[...]
    tab_dma.wait()

    def fill(buf, c):
        cbase = base + c * CHUNK
        conds = [jnp.full((LANES,), cbase + r, jnp.int32) < ntxt
                 for r in range(CHUNK)]

        def body(s, carry):
            off = s * LANES
            e0 = tab_v[TEXT_ID, pl.ds(off, LANES)]
            e3 = tab_v[VISUAL_ID, pl.ds(off, LANES)]
            for r in range(CHUNK):
                buf[r, 0, pl.ds(off, LANES)] = jnp.where(conds[r], e0, e3)
            return carry
        lax.fori_loop(0, SLICES, body, 0)

    bufs = (buf0, buf1)
    sems = (sem0, sem1)
    handles = [None, None]
    for c in range(NCHUNK):
        b = c % 2
        if handles[b] is not None:
            handles[b].wait()
        fill(bufs[b], c)
        handles[b] = pltpu.async_copy(
            bufs[b], out_hbm.at[pl.ds(base + c * CHUNK, CHUNK)], sems[b])
    for h in handles:
        if h is not None:
            h.wait()


def kernel(x, num_frame, embedding):
    L, N, D = x.shape
    num_txt = jnp.full((LANES,), L - num_frame, dtype=jnp.int32)
    return _emb_lookup(embedding, num_txt)
